# Initial kernel scaffold; baseline (speedup 1.0000x reference)
import functools

import jax
import jax.numpy as jnp
from jax import lax
from jax.experimental import pallas as pl
from jax.experimental.pallas import tpu as pltpu
from jax.experimental.pallas import tpu_sc as plsc

VOCAB = 100000
EMBED = 64
BATCH = 16384
SEQ = 20

NUM_CORES = 2
NUM_SUBCORES = 16
NW = NUM_CORES * NUM_SUBCORES

TOTAL = BATCH * SEQ
B_PER_W = TOTAL // NW
CHUNK = 1024
NSTEP = B_PER_W // CHUNK

assert TOTAL % NW == 0 and B_PER_W % CHUNK == 0 and CHUNK % 8 == 0

_mesh = plsc.VectorSubcoreMesh(core_axis_name="c", subcore_axis_name="s")


@functools.partial(
    pl.kernel,
    mesh=_mesh,
    out_type=jax.ShapeDtypeStruct((TOTAL, EMBED), jnp.float32),
    scratch_types=[
        pltpu.VMEM((CHUNK,), jnp.int32),
        pltpu.VMEM((CHUNK, EMBED), jnp.float32),
        pltpu.SemaphoreType.DMA,
    ],
)
def _gather_kernel(table_hbm, q_hbm, out_hbm, idx_v, rows_v, sem):
    wid = lax.axis_index("s") * NUM_CORES + lax.axis_index("c")
    base = wid * B_PER_W

    def step(g, carry):
        off = base + g * CHUNK
        pltpu.sync_copy(q_hbm.at[pl.ds(off, CHUNK)], idx_v)
        pltpu.async_copy(table_hbm.at[idx_v], rows_v, sem).wait()
        pltpu.sync_copy(rows_v, out_hbm.at[pl.ds(off, CHUNK)])
        return carry

    lax.fori_loop(0, NSTEP, step, 0)


def kernel(q, q_len, table):
    del q_len
    qflat = q.reshape(TOTAL).astype(jnp.int32)
    out = _gather_kernel(table, qflat)
    return out.reshape(BATCH, SEQ, EMBED)


# speedup vs baseline: 4.0946x; 4.0946x over previous
import functools

import jax
import jax.numpy as jnp
from jax import lax
from jax.experimental import pallas as pl
from jax.experimental.pallas import tpu as pltpu
from jax.experimental.pallas import tpu_sc as plsc

VOCAB = 100000
EMBED = 64
BATCH = 16384
SEQ = 20

NUM_CORES = 2
NUM_SUBCORES = 16
NW = NUM_CORES * NUM_SUBCORES

TOTAL = BATCH * SEQ
B_PER_W = TOTAL // NW
CHUNK = 1024
NSTEP = B_PER_W // CHUNK

assert TOTAL % NW == 0 and B_PER_W % CHUNK == 0 and CHUNK % 8 == 0

_mesh = plsc.VectorSubcoreMesh(core_axis_name="c", subcore_axis_name="s")


@functools.partial(
    pl.kernel,
    mesh=_mesh,
    out_type=jax.ShapeDtypeStruct((TOTAL, EMBED), jnp.float32),
    scratch_types=[
        pltpu.VMEM((CHUNK,), jnp.int32),
        pltpu.VMEM((CHUNK, EMBED), jnp.float32),
        pltpu.SemaphoreType.DMA,
    ],
    compiler_params=pltpu.CompilerParams(use_tc_tiling_on_sc=False),
)
def _gather_kernel(table_hbm, q_hbm, out_hbm, idx_v, rows_v, sem):
    wid = lax.axis_index("s") * NUM_CORES + lax.axis_index("c")
    base = wid * B_PER_W

    def step(g, carry):
        off = base + g * CHUNK
        pltpu.sync_copy(q_hbm.at[pl.ds(off, CHUNK)], idx_v)
        pltpu.async_copy(table_hbm.at[idx_v], rows_v, sem).wait()
        pltpu.sync_copy(rows_v, out_hbm.at[pl.ds(off, CHUNK)])
        return carry

    lax.fori_loop(0, NSTEP, step, 0)


def kernel(q, q_len, table):
    del q_len
    qflat = q.reshape(TOTAL).astype(jnp.int32)
    out = _gather_kernel(table, qflat)
    return out.reshape(BATCH, SEQ, EMBED)


# speedup vs baseline: 4.1976x; 1.0251x over previous
import functools

import jax
import jax.numpy as jnp
from jax import lax
from jax.experimental import pallas as pl
from jax.experimental.pallas import tpu as pltpu
from jax.experimental.pallas import tpu_sc as plsc

VOCAB = 100000
EMBED = 64
BATCH = 16384
SEQ = 20

NUM_CORES = 2
NUM_SUBCORES = 16
NW = NUM_CORES * NUM_SUBCORES

TOTAL = BATCH * SEQ
B_PER_W = TOTAL // NW
CHUNK = 512
NSTEP = B_PER_W // CHUNK
NBUF = 2

assert TOTAL % NW == 0 and B_PER_W % CHUNK == 0 and CHUNK % 8 == 0

_mesh = plsc.VectorSubcoreMesh(core_axis_name="c", subcore_axis_name="s")


@functools.partial(
    pl.kernel,
    mesh=_mesh,
    out_type=jax.ShapeDtypeStruct((TOTAL, EMBED), jnp.float32),
    scratch_types=[
        pltpu.VMEM((B_PER_W,), jnp.int32),
        [pltpu.VMEM((CHUNK, EMBED), jnp.float32) for _ in range(NBUF)],
        [pltpu.SemaphoreType.DMA for _ in range(NBUF)],
        [pltpu.SemaphoreType.DMA for _ in range(NBUF)],
    ],
    compiler_params=pltpu.CompilerParams(use_tc_tiling_on_sc=False),
)
def _gather_kernel(table_hbm, q_hbm, out_hbm, idx_all, rows, gsem, osem):
    wid = lax.axis_index("s") * NUM_CORES + lax.axis_index("c")
    base = wid * B_PER_W

    pltpu.sync_copy(q_hbm.at[pl.ds(base, B_PER_W)], idx_all)

    def fire_gather(g):
        idx_slice = idx_all.at[pl.ds(g * CHUNK, CHUNK)]
        return pltpu.async_copy(table_hbm.at[idx_slice], rows[g % NBUF],
                                gsem[g % NBUF])

    def fire_write(g):
        return pltpu.async_copy(rows[g % NBUF],
                                out_hbm.at[pl.ds(base + g * CHUNK, CHUNK)],
                                osem[g % NBUF])

    gathers = {0: fire_gather(0)}
    writes = {}
    for g in range(NSTEP):
        if g + 1 < NSTEP:
            prev_w = g + 1 - NBUF
            if prev_w >= 0:
                writes.pop(prev_w).wait()
            gathers[g + 1] = fire_gather(g + 1)
        gathers.pop(g).wait()
        writes[g] = fire_write(g)
    for g in sorted(writes):
        writes.pop(g).wait()


def kernel(q, q_len, table):
    del q_len
    qflat = q.reshape(TOTAL).astype(jnp.int32)
    out = _gather_kernel(table, qflat)
    return out.reshape(BATCH, SEQ, EMBED)


# speedup vs baseline: 6.7775x; 1.6146x over previous
import functools

import jax
import jax.numpy as jnp
from jax import lax
from jax.experimental import pallas as pl
from jax.experimental.pallas import tpu as pltpu
from jax.experimental.pallas import tpu_sc as plsc

VOCAB = 100000
EMBED = 64
BATCH = 16384
SEQ = 20
SEQ_PAD = 24
LANE_PAD = 128

NUM_CORES = 2
NUM_SUBCORES = 16
NW = NUM_CORES * NUM_SUBCORES

TOTAL = BATCH * SEQ
B_PER_W = TOTAL // NW
BATCH_PER_W = BATCH // NW
CHUNK_B = 32
CHUNK = CHUNK_B * SEQ
NSTEP = BATCH_PER_W // CHUNK_B
NBUF = 2

assert BATCH % NW == 0 and BATCH_PER_W % CHUNK_B == 0 and CHUNK % 8 == 0

_mesh = plsc.VectorSubcoreMesh(core_axis_name="c", subcore_axis_name="s")


@functools.partial(
    pl.kernel,
    mesh=_mesh,
    out_type=jax.ShapeDtypeStruct((BATCH, SEQ_PAD, LANE_PAD), jnp.float32),
    scratch_types=[
        pltpu.VMEM((B_PER_W,), jnp.int32),
        [pltpu.VMEM((CHUNK, EMBED), jnp.float32) for _ in range(NBUF)],
        [pltpu.SemaphoreType.DMA for _ in range(NBUF)],
        [pltpu.SemaphoreType.DMA for _ in range(NBUF)],
    ],
    compiler_params=pltpu.CompilerParams(use_tc_tiling_on_sc=False),
)
def _gather_kernel(table_hbm, q_hbm, out_hbm, idx_all, rows, gsem, osem):
    wid = lax.axis_index("s") * NUM_CORES + lax.axis_index("c")
    tok_base = wid * B_PER_W
    batch_base = wid * BATCH_PER_W

    pltpu.sync_copy(q_hbm.at[pl.ds(tok_base, B_PER_W)], idx_all)

    def fire_gather(g):
        idx_slice = idx_all.at[pl.ds(g * CHUNK, CHUNK)]
        return pltpu.async_copy(table_hbm.at[idx_slice], rows[g % NBUF],
                                gsem[g % NBUF])

    def fire_writes(g):
        buf = rows[g % NBUF]
        sem = osem[g % NBUF]
        copies = []
        for j in range(CHUNK_B):
            b = batch_base + g * CHUNK_B + j
            dst = out_hbm.at[b, pl.ds(0, SEQ), pl.ds(0, EMBED)]
            copies.append(
                pltpu.async_copy(buf.at[pl.ds(j * SEQ, SEQ)], dst, sem))
        return copies

    gathers = {0: fire_gather(0)}
    writes = {}
    for g in range(NSTEP):
        if g + 1 < NSTEP:
            prev_w = g + 1 - NBUF
            if prev_w >= 0:
                for c in writes.pop(prev_w):
                    c.wait()
            gathers[g + 1] = fire_gather(g + 1)
        gathers.pop(g).wait()
        writes[g] = fire_writes(g)
    for g in sorted(writes):
        for c in writes.pop(g):
            c.wait()


def kernel(q, q_len, table):
    del q_len
    qflat = q.reshape(TOTAL).astype(jnp.int32)
    padded = _gather_kernel(table, qflat)
    return padded[:, :SEQ, :EMBED]


# speedup vs baseline: 7.0642x; 1.0423x over previous
import functools

import jax
import jax.numpy as jnp
from jax import lax
from jax.experimental import pallas as pl
from jax.experimental.pallas import tpu as pltpu
from jax.experimental.pallas import tpu_sc as plsc

VOCAB = 100000
EMBED = 64
BATCH = 16384
SEQ = 20
SEQ_PAD = 24
LANE_PAD = 128

NUM_CORES = 2
NUM_SUBCORES = 16
NW = NUM_CORES * NUM_SUBCORES

TOTAL = BATCH * SEQ
B_PER_W = TOTAL // NW
BATCH_PER_W = BATCH // NW
CHUNK_B = 32
CHUNK = CHUNK_B * SEQ
NSTEP = BATCH_PER_W // CHUNK_B
NBUF = 2

assert BATCH % NW == 0 and BATCH_PER_W % CHUNK_B == 0 and CHUNK % 8 == 0

_mesh = plsc.VectorSubcoreMesh(core_axis_name="c", subcore_axis_name="s")


@functools.partial(
    pl.kernel,
    mesh=_mesh,
    out_type=jax.ShapeDtypeStruct((BATCH, SEQ_PAD, LANE_PAD), jnp.float32),
    scratch_types=[
        pltpu.VMEM((B_PER_W,), jnp.int32),
        [pltpu.VMEM((CHUNK, EMBED), jnp.float32) for _ in range(NBUF)],
        [pltpu.SemaphoreType.DMA for _ in range(NBUF)],
        [pltpu.SemaphoreType.DMA for _ in range(NBUF)],
    ],
    compiler_params=pltpu.CompilerParams(use_tc_tiling_on_sc=False),
)
def _gather_kernel(table_hbm, q_hbm, out_hbm, idx_all, rows, gsem, osem):
    wid = lax.axis_index("s") * NUM_CORES + lax.axis_index("c")
    tok_base = wid * B_PER_W
    batch_base = wid * BATCH_PER_W

    pltpu.sync_copy(q_hbm.at[pl.ds(tok_base, B_PER_W)], idx_all)

    def fire_gather(g):
        idx_slice = idx_all.at[pl.ds(g * CHUNK, CHUNK)]
        return pltpu.async_copy(table_hbm.at[idx_slice], rows[g % NBUF],
                                gsem[g % NBUF])

    def fire_writes(g):
        buf = rows[g % NBUF]
        sem = osem[g % NBUF]
        copies = []
        for j in range(CHUNK_B):
            b = batch_base + g * CHUNK_B + j
            dst = out_hbm.at[b, pl.ds(0, SEQ), pl.ds(0, EMBED)]
            copies.append(
                pltpu.async_copy(buf.at[pl.ds(j * SEQ, SEQ)], dst, sem))
        return copies

    gathers = {0: fire_gather(0)}
    writes = {}
    for g in range(NSTEP):
        if g + 1 < NSTEP:
            prev_w = g + 1 - NBUF
            if prev_w >= 0:
                for c in writes.pop(prev_w):
                    c.wait()
            gathers[g + 1] = fire_gather(g + 1)
        gathers.pop(g).wait()
        writes[g] = fire_writes(g)
    for g in sorted(writes):
        for c in writes.pop(g):
            c.wait()


def kernel(q, q_len, table):
    del q_len
    qflat2 = q.reshape(TOTAL).astype(jnp.int32) * 2
    table2 = jnp.pad(table, ((0, 0), (0, LANE_PAD - EMBED))).reshape(
        2 * VOCAB, EMBED)
    padded = _gather_kernel(table2, qflat2)
    return padded[:, :SEQ, :EMBED]
